# baseline (speedup 1.0000x reference)
import jax
import jax.numpy as jnp
from jax import lax
from jax.experimental import pallas as pl
from jax.experimental.pallas import tpu as pltpu
from jax.experimental.pallas import tpu_sc as plsc

BATCH = 16384
DIM = 128
LANES = 16
CHUNK = 128
N_WORKERS = 32
ROWS_PER_W = BATCH // N_WORKERS
N_CHUNKS = ROWS_PER_W // CHUNK

TC_BLOCK = 1024
TC_GRID = BATCH // TC_BLOCK
SUBS = TC_BLOCK // DIM


def _sc_gather_body(j_hbm, ep_hbm, lo_hbm, sc_hbm,
                    idx_v, idxp1_v, lo_v, hi_v, sc_v, sem_g):
    wid = lax.axis_index("s") * 2 + lax.axis_index("c")
    base = wid * ROWS_PER_W

    gathers = []
    for c in range(N_CHUNKS):
        pltpu.sync_copy(j_hbm.at[pl.ds(base + c * CHUNK, CHUNK)], idx_v.at[c])
        for v in range(CHUNK // LANES):
            s = pl.ds(v * LANES, LANES)
            idxp1_v[c, s] = idx_v[c, s] + 1
        gathers.append(pltpu.async_copy(ep_hbm.at[idx_v.at[c]], lo_v.at[c], sem_g))
        gathers.append(pltpu.async_copy(ep_hbm.at[idxp1_v.at[c]], hi_v.at[c], sem_g))
    for g in gathers:
        g.wait()
    for c in range(N_CHUNKS):
        for v in range(CHUNK // LANES):
            s = pl.ds(v * LANES, LANES)
            sc_v[c, s] = hi_v[c, s] - lo_v[c, s]
    pltpu.sync_copy(lo_v, lo_hbm.at[pl.ds(wid * N_CHUNKS, N_CHUNKS), :])
    pltpu.sync_copy(sc_v, sc_hbm.at[pl.ds(wid * N_CHUNKS, N_CHUNKS), :])


def _tc_affine_body(z_ref, lo_ref, sc_ref, o_ref):
    lo_t = jnp.transpose(lo_ref[...])
    sc_t = jnp.transpose(sc_ref[...])
    for s in range(SUBS):
        rows = pl.ds(s * DIM, DIM)
        lo_col = lo_t[:, s:s + 1]
        sc_col = sc_t[:, s:s + 1]
        o_ref[rows, :] = z_ref[rows, :] * sc_col + lo_col


@jax.jit
def _shift_mapper(z, j_flat, endpoints):
    mesh = plsc.VectorSubcoreMesh(core_axis_name="c", subcore_axis_name="s")
    gather_fn = pl.kernel(
        _sc_gather_body,
        mesh=mesh,
        out_type=(
            jax.ShapeDtypeStruct((DIM, DIM), jnp.float32),
            jax.ShapeDtypeStruct((DIM, DIM), jnp.float32),
        ),
        scratch_types=[
            pltpu.VMEM((N_CHUNKS, CHUNK), jnp.int32),
            pltpu.VMEM((N_CHUNKS, CHUNK), jnp.int32),
            pltpu.VMEM((N_CHUNKS, CHUNK), jnp.float32),
            pltpu.VMEM((N_CHUNKS, CHUNK), jnp.float32),
            pltpu.VMEM((N_CHUNKS, CHUNK), jnp.float32),
            pltpu.SemaphoreType.DMA,
        ],
        compiler_params=pltpu.CompilerParams(needs_layout_passes=False),
    )
    lo_arr, sc_arr = gather_fn(j_flat, endpoints)

    out = pl.pallas_call(
        _tc_affine_body,
        grid=(TC_GRID,),
        in_specs=[
            pl.BlockSpec((TC_BLOCK, DIM), lambda i: (i, 0)),
            pl.BlockSpec((SUBS, DIM), lambda i: (i, 0)),
            pl.BlockSpec((SUBS, DIM), lambda i: (i, 0)),
        ],
        out_specs=pl.BlockSpec((TC_BLOCK, DIM), lambda i: (i, 0)),
        out_shape=jax.ShapeDtypeStruct((BATCH, DIM), jnp.float32),
    )(z, lo_arr, sc_arr)
    return out


def kernel(z, j, endpoints):
    j_flat = j.reshape(-1).astype(jnp.int32)
    return _shift_mapper(z, j_flat, endpoints)


# speedup vs baseline: 1.3561x; 1.3561x over previous
import jax
import jax.numpy as jnp
from jax import lax
from jax.experimental import pallas as pl
from jax.experimental.pallas import tpu as pltpu
from jax.experimental.pallas import tpu_sc as plsc

BATCH = 16384
DIM = 128
LANES = 16
N_WORKERS = 32
ROWS_PER_W = BATCH // N_WORKERS
CHUNK = 256
N_CHUNKS = ROWS_PER_W // CHUNK
GSLICE = 128


def _sc_body(z_hbm, j_hbm, ep_hbm, out_hbm,
             idx_all, idxp1_all, lo_all, hi_all,
             z_b0, z_b1, sem_z0, sem_z1, sem_o0, sem_o1, sem_g):
    wid = lax.axis_index("s") * 2 + lax.axis_index("c")
    base = wid * ROWS_PER_W
    z_b = [z_b0, z_b1]
    sem_z = [sem_z0, sem_z1]
    sem_o = [sem_o0, sem_o1]

    fills = [
        pltpu.async_copy(
            z_hbm.at[pl.ds(base + k * CHUNK, CHUNK), :], z_b[k], sem_z[k])
        for k in range(N_CHUNKS)
    ]
    pltpu.sync_copy(j_hbm.at[pl.ds(base, ROWS_PER_W)], idx_all)
    for v in range(ROWS_PER_W // LANES):
        s = pl.ds(v * LANES, LANES)
        idxp1_all[s] = idx_all[s] + 1
    gathers = []
    for g in range(ROWS_PER_W // GSLICE):
        s = pl.ds(g * GSLICE, GSLICE)
        gathers.append(
            pltpu.async_copy(ep_hbm.at[idx_all.at[s]], lo_all.at[s], sem_g))
        gathers.append(
            pltpu.async_copy(ep_hbm.at[idxp1_all.at[s]], hi_all.at[s], sem_g))
    for g in gathers:
        g.wait()

    drains = []
    for k in range(N_CHUNKS):
        off = k * CHUNK
        fills[k].wait()
        zb = z_b[k]

        def grp_body(gi, _):
            o = off + gi * LANES
            lo_vec = lo_all[pl.ds(o, LANES)]
            hi_vec = hi_all[pl.ds(o, LANES)]
            sc_vec = hi_vec - lo_vec
            for r in range(LANES):
                lo_s = lo_vec[r]
                sc_s = sc_vec[r]
                row = gi * LANES + r
                for v in range(DIM // LANES):
                    s = pl.ds(v * LANES, LANES)
                    zb[row, s] = zb[row, s] * sc_s + lo_s
            return 0

        lax.fori_loop(0, CHUNK // LANES, grp_body, 0)
        drains.append(pltpu.async_copy(
            zb, out_hbm.at[pl.ds(base + off, CHUNK), :], sem_o[k]))
    for d in drains:
        d.wait()


@jax.jit
def _shift_mapper_sc(z, j_flat, endpoints):
    mesh = plsc.VectorSubcoreMesh(core_axis_name="c", subcore_axis_name="s")
    kfn = pl.kernel(
        _sc_body,
        mesh=mesh,
        out_type=jax.ShapeDtypeStruct((BATCH, DIM), jnp.float32),
        scratch_types=[
            pltpu.VMEM((ROWS_PER_W,), jnp.int32),
            pltpu.VMEM((ROWS_PER_W,), jnp.int32),
            pltpu.VMEM((ROWS_PER_W,), jnp.float32),
            pltpu.VMEM((ROWS_PER_W,), jnp.float32),
            pltpu.VMEM((CHUNK, DIM), jnp.float32),
            pltpu.VMEM((CHUNK, DIM), jnp.float32),
            pltpu.SemaphoreType.DMA,
            pltpu.SemaphoreType.DMA,
            pltpu.SemaphoreType.DMA,
            pltpu.SemaphoreType.DMA,
            pltpu.SemaphoreType.DMA,
        ],
        compiler_params=pltpu.CompilerParams(needs_layout_passes=False),
    )
    return kfn(z, j_flat, endpoints)


def kernel(z, j, endpoints):
    j_flat = j.reshape(-1).astype(jnp.int32)
    return _shift_mapper_sc(z, j_flat, endpoints)
